# baseline (speedup 1.0000x reference)
import functools
import jax
import jax.numpy as jnp
from jax import lax
from jax.experimental import pallas as pl
from jax.experimental.pallas import tpu as pltpu
from jax.experimental.pallas import tpu_sc as plsc

S = 2048
B = 4
D = 1024
NC = 2
NS = 16
NW = NC * NS
ROWS_W = S // NW
CHS = 4
NCH = ROWS_W // CHS
NV = D // 16
NSLOT = 4
NEA = ROWS_W // 8


def _sc_body(x_hbm, e_hbm, o_hbm, xbuf, eabuf, ebbuf, xsem, easem, ebsem,
             osem):
    wid = lax.axis_index("s") * NC + lax.axis_index("c")
    s0 = wid * ROWS_W

    def in_x(c):
        return pltpu.make_async_copy(
            x_hbm.at[pl.ds(s0 + c * CHS, CHS)], xbuf.at[c % NSLOT],
            xsem.at[c % NSLOT])

    def in_ea(k):
        return pltpu.make_async_copy(
            e_hbm.at[pl.ds(s0 + 8 * k, 8)], eabuf.at[k % NSLOT],
            easem.at[k % NSLOT])

    def in_eb():
        return pltpu.make_async_copy(
            e_hbm.at[pl.ds(s0 + ROWS_W, 1)], ebbuf, ebsem)

    def out_o(c):
        return pltpu.make_async_copy(
            xbuf.at[c % NSLOT], o_hbm.at[pl.ds(s0 + c * CHS, CHS)],
            osem.at[c % NSLOT])

    in_x(0).start()
    in_x(1).start()
    in_x(2).start()
    in_x(3).start()
    in_ea(0).start()
    in_ea(1).start()
    in_eb().start()

    for c in range(NCH):
        if c >= 3 and c + 1 < NCH:
            out_o(c - 3).wait()
            in_x(c + 1).start()
        in_x(c).wait()
        k = c // 2
        if c == 0:
            in_ea(0).wait()
        elif c % 2 == 1 and k + 1 < NEA:
            in_ea(k + 1).wait()
        if c % 2 == 0 and k + 2 < NEA:
            in_ea(k + 2).start()
        if c == NCH - 1:
            in_eb().wait()

        def dv_body(dv, _, c=c, k=k):
            for u in range(2):
                dd = pl.ds((dv * 2 + u) * 16, 16)
                for sl in range(CHS):
                    off = (c % 2) * CHS + sl + 1
                    if off < 8:
                        e = eabuf[k % NSLOT, off, dd]
                    elif k + 1 < NEA:
                        e = eabuf[(k + 1) % NSLOT, 0, dd]
                    else:
                        e = ebbuf[0, dd]
                    for b in range(B):
                        xbuf[c % NSLOT, sl, b, dd] = (
                            xbuf[c % NSLOT, sl, b, dd] + e)
            return 0

        lax.fori_loop(0, NV // 2, dv_body, 0)
        out_o(c).start()

    out_o(NCH - 4).wait()
    out_o(NCH - 3).wait()
    out_o(NCH - 2).wait()
    out_o(NCH - 1).wait()


_sc_kernel = functools.partial(
    pl.kernel,
    out_type=jax.ShapeDtypeStruct((S, B, D), jnp.float32),
    mesh=plsc.VectorSubcoreMesh(core_axis_name="c", subcore_axis_name="s"),
    scratch_types=[
        pltpu.VMEM((NSLOT, CHS, B, D), jnp.float32),
        pltpu.VMEM((NSLOT, 8, D), jnp.float32),
        pltpu.VMEM((1, D), jnp.float32),
        pltpu.SemaphoreType.DMA((NSLOT,)),
        pltpu.SemaphoreType.DMA((NSLOT,)),
        pltpu.SemaphoreType.DMA,
        pltpu.SemaphoreType.DMA((NSLOT,)),
    ],
)(_sc_body)


def kernel(inputs, pos_emb):
    return _sc_kernel(inputs, pos_emb)


# speedup vs baseline: 2.0432x; 2.0432x over previous
import functools
import jax
import jax.numpy as jnp
from jax import lax
from jax.experimental import pallas as pl
from jax.experimental.pallas import tpu as pltpu
from jax.experimental.pallas import tpu_sc as plsc

S = 2048
B = 4
D = 1024
NC = 2
NS = 16
NW = NC * NS
ROWS_W = S // NW
CHS = 4
NCH = ROWS_W // CHS
NV = D // 16
NSLOT = 4
NSE = 5
NEA = ROWS_W // 8


def _sc_body(x_hbm, e_hbm, o_hbm, xbuf, eabuf, ebbuf, xsem, easem, ebsem,
             osem):
    wid = lax.axis_index("s") * NC + lax.axis_index("c")
    s0 = wid * ROWS_W

    def in_x(c):
        slot = c % NSLOT
        return pltpu.make_async_copy(
            x_hbm.at[pl.ds(s0 + c * CHS, CHS)], xbuf.at[slot], xsem.at[slot])

    def in_ea(k):
        slot = k % NSE
        return pltpu.make_async_copy(
            e_hbm.at[pl.ds(s0 + 8 * k, 8)], eabuf.at[slot], easem.at[slot])

    def in_eb():
        return pltpu.make_async_copy(
            e_hbm.at[pl.ds(s0 + ROWS_W, 1)], ebbuf, ebsem)

    def out_o(c):
        slot = c % NSLOT
        return pltpu.make_async_copy(
            xbuf.at[slot], o_hbm.at[pl.ds(s0 + c * CHS, CHS)], osem.at[slot])

    in_x(0).start()
    in_x(1).start()
    in_x(2).start()
    in_x(3).start()
    in_ea(0).start()
    in_ea(1).start()
    in_eb().start()

    def chunk_body(c, _):
        k = c // 2

        @pl.when(jnp.logical_and(c >= 3, c < NCH - 1))
        def _():
            out_o(c - 3).wait()
            in_x(c + 1).start()

        in_x(c).wait()

        @pl.when(c == 0)
        def _():
            in_ea(0).wait()

        @pl.when(jnp.logical_and(c % 2 == 1, k + 1 < NEA))
        def _():
            in_ea(k + 1).wait()

        @pl.when(jnp.logical_and(c % 2 == 0, k + 2 < NEA))
        def _():
            in_ea(k + 2).start()

        @pl.when(c == 2 * NEA - 8)
        def _():
            in_eb().wait()

            def cp(dv, _):
                dd = pl.ds(dv * 16, 16)
                eabuf[NEA % NSE, 0, dd] = ebbuf[0, dd]
                return 0

            lax.fori_loop(0, NV, cp, 0)

        def dv_body(dv, _):
            dd = pl.ds(dv * 16, 16)
            odd = c % 2
            row0 = odd * CHS
            xs = c % NSLOT
            for sl in range(CHS - 1):
                e = eabuf[k % NSE, row0 + sl + 1, dd]
                for b in range(B):
                    xbuf[xs, sl, b, dd] = xbuf[xs, sl, b, dd] + e
            e3 = eabuf[((c + 1) // 2) % NSE, (1 - odd) * CHS, dd]
            for b in range(B):
                xbuf[xs, CHS - 1, b, dd] = xbuf[xs, CHS - 1, b, dd] + e3
            return 0

        lax.fori_loop(0, NV, dv_body, 0)
        out_o(c).start()
        return 0

    lax.fori_loop(0, NCH, chunk_body, 0)

    for t in range(NCH - 4, NCH):
        out_o(t).wait()


_sc_kernel = functools.partial(
    pl.kernel,
    out_type=jax.ShapeDtypeStruct((S, B, D), jnp.float32),
    mesh=plsc.VectorSubcoreMesh(core_axis_name="c", subcore_axis_name="s"),
    scratch_types=[
        pltpu.VMEM((NSLOT, CHS, B, D), jnp.float32),
        pltpu.VMEM((NSE, 8, D), jnp.float32),
        pltpu.VMEM((1, D), jnp.float32),
        pltpu.SemaphoreType.DMA((NSLOT,)),
        pltpu.SemaphoreType.DMA((NSE,)),
        pltpu.SemaphoreType.DMA,
        pltpu.SemaphoreType.DMA((NSLOT,)),
    ],
)(_sc_body)


def kernel(inputs, pos_emb):
    return _sc_kernel(inputs, pos_emb)
